# Initial kernel scaffold; baseline (speedup 1.0000x reference)
import jax
import jax.numpy as jnp
from jax.experimental import pallas as pl


def kernel(x, edge_index, batch, W1, b1, W2, b2, mu_W, mu_b, lv_W, lv_b, D1_W, D1_b, D2_W, D2_b, D3_W, D3_b):
    raise NotImplementedError("write your pallas kernel here")


# speedup vs baseline: 12.9572x; 12.9572x over previous
import functools

import jax
import jax.numpy as jnp
from jax import lax
from jax.experimental import pallas as pl
from jax.experimental.pallas import tpu as pltpu
from jax.experimental.pallas import tpu_sc as plsc

N_NODES = 10000
N_EDGES = 320000
CH = 128
LAT = 64
NUM_GRAPHS = 64
MAX_NODES = 64
OUT_SIZE = MAX_NODES * (MAX_NODES - 1) // 2

NC = 2
NS = 16
K = 80
STRIPE = 640


def _sc_degree_kernel(E, NPAD):
    per_core = E // NC
    per_tile = per_core // NS
    iters = per_tile // K
    rows = NPAD // 128
    rows_per_tile = rows // NS
    mesh = plsc.VectorSubcoreMesh(core_axis_name="c", subcore_axis_name="s")

    @functools.partial(
        pl.kernel, mesh=mesh,
        out_type=jax.ShapeDtypeStruct((NC, rows, 128), jnp.float32),
        scratch_types=[
            pltpu.VMEM((K,), jnp.int32),
            pltpu.VMEM((K,), jnp.float32),
            pltpu.VMEM((128,), jnp.float32),
            pltpu.VMEM_SHARED((NPAD,), jnp.float32),
            pltpu.SemaphoreType.DMA,
        ],
    )
    def deg_kernel(dst_hbm, out_hbm, idx_v, ones_v, tmp_v, acc_sh, sem):
        c = lax.axis_index("c")
        s = lax.axis_index("s")
        for j in range(K // 16):
            ones_v[pl.ds(j * 16, 16)] = jnp.full((16,), 1.0, jnp.float32)
        for j in range(8):
            tmp_v[pl.ds(j * 16, 16)] = jnp.zeros((16,), jnp.float32)
        for t in range(rows_per_tile):
            row = s * rows_per_tile + t
            pltpu.sync_copy(tmp_v, acc_sh.at[pl.ds(row * 128, 128)])

        plsc.subcore_barrier()
        base = c * per_core + s * per_tile

        def body(i, carry):
            pltpu.sync_copy(dst_hbm.at[pl.ds(base + i * K, K)], idx_v)
            pltpu.sync_copy(ones_v, acc_sh.at[idx_v], add=True)
            return carry

        lax.fori_loop(0, iters, body, 0)
        plsc.subcore_barrier()
        for t in range(rows_per_tile):
            row = s * rows_per_tile + t
            pltpu.sync_copy(acc_sh.at[pl.ds(row * 128, 128)], tmp_v)
            pltpu.sync_copy(tmp_v, out_hbm.at[c, row])

    return deg_kernel


def _sc_scatter_kernel(E, N, C):
    per_core = E // NC
    per_tile = per_core // NS
    iters = per_tile // K
    nchunks = STRIPE // K
    mesh = plsc.VectorSubcoreMesh(core_axis_name="c", subcore_axis_name="s")

    @functools.partial(
        pl.kernel, mesh=mesh,
        out_type=jax.ShapeDtypeStruct((NC, N, C), jnp.float32),
        scratch_types=[
            pltpu.VMEM((K,), jnp.int32),
            pltpu.VMEM((K,), jnp.int32),
            pltpu.VMEM((K, C), jnp.float32),
            pltpu.VMEM_SHARED((N, C), jnp.float32),
            pltpu.SemaphoreType.DMA,
        ],
    )
    def scat_kernel(rows_hbm, src_hbm, dst_hbm, out_hbm, si_v, di_v, rows_v,
                    acc_sh, sem):
        c = lax.axis_index("c")
        s = lax.axis_index("s")

        def zbody(i, carry):
            for j in range(C // 16):
                rows_v[i, pl.ds(j * 16, 16)] = jnp.zeros((16,), jnp.float32)
            return carry

        lax.fori_loop(0, K, zbody, 0)
        for t in range(nchunks):
            start = s * STRIPE + t * K

            @pl.when(start < N)
            def _():
                pltpu.sync_copy(rows_v, acc_sh.at[pl.ds(start, K)])

        plsc.subcore_barrier()
        base = c * per_core + s * per_tile

        def body(i, carry):
            off = base + i * K
            pltpu.sync_copy(src_hbm.at[pl.ds(off, K)], si_v)
            pltpu.sync_copy(dst_hbm.at[pl.ds(off, K)], di_v)
            pltpu.async_copy(rows_hbm.at[si_v], rows_v, sem).wait()
            pltpu.sync_copy(rows_v, acc_sh.at[di_v], add=True)
            return carry

        lax.fori_loop(0, iters, body, 0)
        plsc.subcore_barrier()
        for t in range(nchunks):
            start = s * STRIPE + t * K

            @pl.when(start < N)
            def _():
                pltpu.sync_copy(acc_sh.at[pl.ds(start, K)], rows_v)
                pltpu.sync_copy(rows_v, out_hbm.at[c, pl.ds(start, K)])

    return scat_kernel


_R = 1000


def _tc_scale_body(x_ref, w_ref, deg_ref, hp_ref, dinv_ref):
    deg = deg_ref[0] + deg_ref[1] + 1.0
    dinv = lax.rsqrt(jnp.maximum(deg, 1.0))
    h = jnp.dot(x_ref[...], w_ref[...], preferred_element_type=jnp.float32)
    hp_ref[...] = h * dinv
    dinv_ref[...] = dinv


def _tc_scale(x, W1, degp):
    n = x.shape[0]
    grid = (n // _R,)
    return pl.pallas_call(
        _tc_scale_body,
        grid=grid,
        in_specs=[
            pl.BlockSpec((_R, CH), lambda i: (i, 0)),
            pl.BlockSpec((CH, CH), lambda i: (0, 0)),
            pl.BlockSpec((NC, _R, 1), lambda i: (0, i, 0)),
        ],
        out_specs=[
            pl.BlockSpec((_R, CH), lambda i: (i, 0)),
            pl.BlockSpec((_R, 1), lambda i: (i, 0)),
        ],
        out_shape=[
            jax.ShapeDtypeStruct((n, CH), jnp.float32),
            jax.ShapeDtypeStruct((n, 1), jnp.float32),
        ],
    )(x, W1, degp)


def _tc_mid_body(s_ref, hp_ref, dinv_ref, b_ref, w_ref, hp2_ref):
    dinv = dinv_ref[...]
    agg = s_ref[0] + s_ref[1] + hp_ref[...]
    x2 = jax.nn.relu(dinv * agg + b_ref[...])
    h2 = jnp.dot(x2, w_ref[...], preferred_element_type=jnp.float32)
    hp2_ref[...] = h2 * dinv


def _tc_mid(S1, hp1, dinv, b1, W2):
    n = hp1.shape[0]
    grid = (n // _R,)
    return pl.pallas_call(
        _tc_mid_body,
        grid=grid,
        in_specs=[
            pl.BlockSpec((NC, _R, CH), lambda i: (0, i, 0)),
            pl.BlockSpec((_R, CH), lambda i: (i, 0)),
            pl.BlockSpec((_R, 1), lambda i: (i, 0)),
            pl.BlockSpec((1, CH), lambda i: (0, 0)),
            pl.BlockSpec((CH, CH), lambda i: (0, 0)),
        ],
        out_specs=pl.BlockSpec((_R, CH), lambda i: (i, 0)),
        out_shape=jax.ShapeDtypeStruct((n, CH), jnp.float32),
    )(S1, hp1, dinv, b1, W2)


def _tc_pool_body(s_ref, hp_ref, dinv_ref, b_ref, batch_ref, hg_ref,
                  pool_ref, cnt_ref):
    i = pl.program_id(0)

    @pl.when(i == 0)
    def _():
        pool_ref[...] = jnp.zeros_like(pool_ref)
        cnt_ref[...] = jnp.zeros_like(cnt_ref)

    dinv = dinv_ref[...]
    agg = s_ref[0] + s_ref[1] + hp_ref[...]
    h = jax.nn.relu(dinv * agg + b_ref[...])
    gi = lax.broadcasted_iota(jnp.int32, (_R, NUM_GRAPHS), 1)
    ind = jnp.where(gi == batch_ref[...], 1.0, 0.0)
    dn = (((0,), (0,)), ((), ()))
    pool_ref[...] += lax.dot_general(ind, h, dn,
                                     preferred_element_type=jnp.float32)
    cnt_ref[...] += lax.dot_general(ind, jnp.ones((_R, 1), jnp.float32), dn,
                                    preferred_element_type=jnp.float32)

    @pl.when(i == pl.num_programs(0) - 1)
    def _():
        hg_ref[...] = pool_ref[...] / jnp.maximum(cnt_ref[...], 1.0)


def _tc_pool(S2, hp2, dinv, b2, batch2d):
    n = hp2.shape[0]
    grid = (n // _R,)
    return pl.pallas_call(
        _tc_pool_body,
        grid=grid,
        in_specs=[
            pl.BlockSpec((NC, _R, CH), lambda i: (0, i, 0)),
            pl.BlockSpec((_R, CH), lambda i: (i, 0)),
            pl.BlockSpec((_R, 1), lambda i: (i, 0)),
            pl.BlockSpec((1, CH), lambda i: (0, 0)),
            pl.BlockSpec((_R, 1), lambda i: (i, 0)),
        ],
        out_specs=pl.BlockSpec((NUM_GRAPHS, CH), lambda i: (0, 0)),
        out_shape=jax.ShapeDtypeStruct((NUM_GRAPHS, CH), jnp.float32),
        scratch_shapes=[
            pltpu.VMEM((NUM_GRAPHS, CH), jnp.float32),
            pltpu.VMEM((NUM_GRAPHS, 1), jnp.float32),
        ],
    )(S2, hp2, dinv, b2, batch2d)


def _tc_decoder_body(hg_ref, muW_ref, mub_ref, lvW_ref, lvb_ref, eps_ref,
                     d1w_ref, d1b_ref, d2w_ref, d2b_ref, d3w_ref, d3b_ref,
                     adj_ref, mu_ref, lv_ref):
    hg = hg_ref[...]
    mu = jnp.dot(hg, muW_ref[...], preferred_element_type=jnp.float32) + mub_ref[...]
    lv = jnp.dot(hg, lvW_ref[...], preferred_element_type=jnp.float32) + lvb_ref[...]
    mu_ref[...] = mu
    lv_ref[...] = lv
    z = mu + eps_ref[...] * jnp.exp(0.5 * lv)
    p = jax.nn.relu(jnp.dot(z, d1w_ref[...], preferred_element_type=jnp.float32) + d1b_ref[...])
    p = jax.nn.relu(jnp.dot(p, d2w_ref[...], preferred_element_type=jnp.float32) + d2b_ref[...])
    logits = jnp.dot(p, d3w_ref[...], preferred_element_type=jnp.float32) + d3b_ref[...]
    probs = jax.nn.sigmoid(logits)
    adj_ref[...] = jnp.zeros((NUM_GRAPHS, MAX_NODES, MAX_NODES), jnp.float32)
    off = 0
    for r in range(MAX_NODES - 1):
        w = MAX_NODES - 1 - r
        adj_ref[:, r, pl.ds(r + 1, w)] = probs[:, off:off + w]
        off += w
    a = adj_ref[...]
    adj_ref[...] = a + jnp.swapaxes(a, 1, 2)


def _tc_decoder(hg, mu_W, mu_b, lv_W, lv_b, eps, D1_W, D1_b, D2_W, D2_b,
                D3_W, D3_b):
    return pl.pallas_call(
        _tc_decoder_body,
        out_shape=[
            jax.ShapeDtypeStruct((NUM_GRAPHS, MAX_NODES, MAX_NODES), jnp.float32),
            jax.ShapeDtypeStruct((NUM_GRAPHS, LAT), jnp.float32),
            jax.ShapeDtypeStruct((NUM_GRAPHS, LAT), jnp.float32),
        ],
    )(hg, mu_W, mu_b.reshape(1, LAT), lv_W, lv_b.reshape(1, LAT), eps,
      D1_W, D1_b.reshape(1, CH), D2_W, D2_b.reshape(1, CH),
      D3_W, D3_b.reshape(1, OUT_SIZE))


def kernel(x, edge_index, batch, W1, b1, W2, b2, mu_W, mu_b, lv_W, lv_b,
           D1_W, D1_b, D2_W, D2_b, D3_W, D3_b):
    n, c = x.shape
    e = edge_index.shape[1]
    src = edge_index[0].astype(jnp.int32)
    dst = edge_index[1].astype(jnp.int32)
    batch2d = batch.astype(jnp.int32).reshape(n, 1)

    npad = ((n + 2047) // 2048) * 2048
    degp = _sc_degree_kernel(e, npad)(dst)
    degp3 = degp.reshape(NC, npad, 1)[:, :n, :]
    hp1, dinv = _tc_scale(x, W1, degp3)
    S1 = _sc_scatter_kernel(e, n, c)(hp1, src, dst)
    hp2 = _tc_mid(S1, hp1, dinv, b1.reshape(1, CH), W2)
    S2 = _sc_scatter_kernel(e, n, c)(hp2, src, dst)
    hg = _tc_pool(S2, hp2, dinv, b2.reshape(1, CH), batch2d)
    eps = jax.random.normal(jax.random.key(42), (NUM_GRAPHS, LAT), jnp.float32)
    adj, mu, lv = _tc_decoder(hg, mu_W, mu_b, lv_W, lv_b, eps,
                              D1_W, D1_b, D2_W, D2_b, D3_W, D3_b)
    return adj, mu, lv
